# Initial kernel scaffold; baseline (speedup 1.0000x reference)
import jax
import jax.numpy as jnp
from jax.experimental import pallas as pl


def kernel(pts, means3D, opas, u, v, semantics, scales, rot3D):
    raise NotImplementedError("write your pallas kernel here")


# speedup vs baseline: 1.6190x; 1.6190x over previous
import jax
import jax.numpy as jnp
from jax.experimental import pallas as pl
from jax.experimental.pallas import tpu as pltpu

_SCALE_MULT = 0.05
_GRID = 0.005
_RADII_MIN = 1.0

_N_BLK = 1024


def _agg_kernel(pts_ref, pint_ref, r0_ref, r1_ref, r2_ref, ck_ref, w_ref,
                opas_ref, mint_ref, radii_ref, B_ref, out_ref):
    pts = pts_ref[...]
    ck = ck_ref[...]
    w = w_ref[...]

    power = None
    for k, r_ref in enumerate((r0_ref, r1_ref, r2_ref)):
        Pk = jnp.dot(pts, r_ref[...], preferred_element_type=jnp.float32)
        dd = Pk - ck[k:k + 1, :]
        term = dd * dd * w[k:k + 1, :]
        power = term if power is None else power + term

    pint = pint_ref[...]
    mint = mint_ref[...]
    radii = radii_ref[...]
    mask = (
        (jnp.abs(pint[:, 0:1] - mint[0:1, :]) <= radii)
        & (jnp.abs(pint[:, 1:2] - mint[1:2, :]) <= radii)
        & (jnp.abs(pint[:, 2:3] - mint[2:3, :]) <= radii)
    )

    g = jnp.exp(-0.5 * power) * opas_ref[...]
    a = jnp.where(mask, g, 0.0).astype(jnp.bfloat16)
    out_ref[...] = jnp.dot(a, B_ref[...], preferred_element_type=jnp.float32)


def kernel(pts, means3D, opas, u, v, semantics, scales, rot3D):
    pts = pts[0]
    means3D = means3D[0]
    opas = opas[0]
    u = u[0]
    v = v[0]
    semantics = semantics[0]
    scales = scales[0]
    rot3D = rot3D[0]

    n, m = pts.shape[0], means3D.shape[0]
    c = semantics.shape[1]

    pint = jnp.floor(pts / _GRID).astype(jnp.int32)
    mint = jnp.floor(means3D / _GRID).astype(jnp.int32).T
    radii = jnp.maximum(
        jnp.ceil(scales.max(axis=-1) * _SCALE_MULT / _GRID), _RADII_MIN
    ).astype(jnp.int32)[None, :]

    q = rot3D / jnp.linalg.norm(rot3D, axis=-1, keepdims=True)
    qw, qx, qy, qz = q[:, 0], q[:, 1], q[:, 2], q[:, 3]
    r0 = jnp.stack([1 - 2 * (qy * qy + qz * qz), 2 * (qx * qy - qw * qz),
                    2 * (qx * qz + qw * qy)], axis=-1)
    r1 = jnp.stack([2 * (qx * qy + qw * qz), 1 - 2 * (qx * qx + qz * qz),
                    2 * (qy * qz - qw * qx)], axis=-1)
    r2 = jnp.stack([2 * (qx * qz - qw * qy), 2 * (qy * qz + qw * qx),
                    1 - 2 * (qx * qx + qy * qy)], axis=-1)
    R = jnp.stack([r0, r1, r2], axis=-2)
    cks = jnp.stack([jnp.sum(means3D * R[:, :, k], axis=-1)
                     for k in range(3)], axis=0)
    s2 = scales * scales + 1e-8
    ws = (1.0 / s2).T
    rT_bf = [R[:, :, k].T.astype(jnp.bfloat16) for k in range(3)]

    B = jnp.concatenate(
        [semantics * v[:, None], u[:, None], jnp.ones((m, 1), jnp.float32)],
        axis=1,
    ).astype(jnp.bfloat16)

    grid = (n + _N_BLK - 1) // _N_BLK
    full = lambda i: (0, 0)
    out = pl.pallas_call(
        _agg_kernel,
        grid=(grid,),
        in_specs=[
            pl.BlockSpec((_N_BLK, 3), lambda i: (i, 0)),
            pl.BlockSpec((_N_BLK, 3), lambda i: (i, 0)),
            pl.BlockSpec((3, m), full),
            pl.BlockSpec((3, m), full),
            pl.BlockSpec((3, m), full),
            pl.BlockSpec((3, m), full),
            pl.BlockSpec((3, m), full),
            pl.BlockSpec((1, m), full),
            pl.BlockSpec((3, m), full),
            pl.BlockSpec((1, m), full),
            pl.BlockSpec((m, c + 2), full),
        ],
        out_specs=pl.BlockSpec((_N_BLK, c + 2), lambda i: (i, 0)),
        out_shape=jax.ShapeDtypeStruct((n, c + 2), jnp.float32),
        compiler_params=pltpu.CompilerParams(
            dimension_semantics=("arbitrary",),
        ),
    )(pts.astype(jnp.bfloat16), pint, *rT_bf, cks, ws, opas[None, :],
      mint, radii, B)

    logits = out[:, :c]
    bin_logits = out[:, c]
    density = out[:, c + 1]
    return logits, bin_logits, density
